# Initial kernel scaffold; baseline (speedup 1.0000x reference)
import jax
import jax.numpy as jnp
from jax.experimental import pallas as pl


def kernel(table, w_d_w, w_d_b, w_e_w, w_e_b, eat_w, eat_b, flat_ids, cu_seqlens, head_ids, tail_ids):
    raise NotImplementedError("write your pallas kernel here")


# speedup vs baseline: 2.7798x; 2.7798x over previous
import functools

import jax
import jax.numpy as jnp
from jax import lax
from jax.experimental import pallas as pl
from jax.experimental.pallas import tpu as pltpu
from jax.experimental.pallas import tpu_sc as plsc

D = 128
NC = 2
NS = 16
NW = NC * NS
GCHUNK = 96


def _make_gather(total_rows: int):
    assert total_rows % (NW * GCHUNK) == 0
    per_w = total_rows // NW
    n_chunks = per_w // GCHUNK
    mesh = plsc.VectorSubcoreMesh(core_axis_name="c", subcore_axis_name="s")

    @functools.partial(
        pl.kernel,
        mesh=mesh,
        out_type=jax.ShapeDtypeStruct((total_rows, D), jnp.float32),
        scratch_types=[
            pltpu.VMEM((GCHUNK,), jnp.int32),
            pltpu.VMEM((GCHUNK, D), jnp.float32),
            pltpu.SemaphoreType.DMA,
        ],
    )
    def gather_kernel(table_hbm, ids_hbm, out_hbm, idx_v, rows_v, sem):
        wid = lax.axis_index("s") * NC + lax.axis_index("c")
        base = wid * per_w

        def chunk(j, carry):
            off = pl.multiple_of(base + j * GCHUNK, 8)
            pltpu.sync_copy(ids_hbm.at[pl.ds(off, GCHUNK)], idx_v)
            pltpu.async_copy(table_hbm.at[idx_v], rows_v, sem).wait()
            pltpu.sync_copy(rows_v, out_hbm.at[pl.ds(off, GCHUNK), :])
            return carry

        lax.fori_loop(0, n_chunks, chunk, 0)

    return gather_kernel


def _tc_body(emb_ref, head_ref, tail_ref, lo_ref, hi_ref, lo_col_ref,
             hi_col_ref, eat_ref, wcat_ref, bias_ref, out_ref, s_scr):
    f32 = jnp.float32
    T = emb_ref.shape[0]
    B = head_ref.shape[0]
    BLK = 512
    NBLK = T // BLK

    lo = lo_ref[...]
    hi = hi_ref[...]
    head = head_ref[...]
    tail = tail_ref[...]
    eat = eat_ref[...]
    bias = bias_ref[...]
    idx0 = lax.broadcasted_iota(jnp.int32, (BLK, B), 0)

    def pass_a(i, den):
        emb = emb_ref[pl.ds(i * BLK, BLK), :]
        s = jnp.exp(jnp.sum(emb * eat, axis=1))
        s_scr[pl.ds(i, 1), :] = s.reshape(1, BLK)
        pos = idx0 + i * BLK
        onehot = jnp.logical_and(pos >= lo, pos < hi).astype(f32)
        return den + jnp.sum(onehot * s[:, None], axis=0, keepdims=True)

    den = lax.fori_loop(0, NBLK, pass_a, jnp.zeros((1, B), f32))

    def pass_b(i, acc):
        emb = emb_ref[pl.ds(i * BLK, BLK), :]
        s = s_scr[pl.ds(i, 1), :].reshape(BLK)
        pos = idx0 + i * BLK
        onehot = jnp.logical_and(pos >= lo, pos < hi).astype(f32)
        att = s / jnp.sum(onehot * den, axis=1)
        h_tok = jnp.dot(onehot, head, preferred_element_type=f32)
        t_tok = jnp.dot(onehot, tail, preferred_element_type=f32)
        en = jnp.sqrt(jnp.sum(emb * emb, axis=1))
        hn = jnp.sqrt(jnp.sum(h_tok * h_tok, axis=1))
        tn = jnp.sqrt(jnp.sum(t_tok * t_tok, axis=1))
        sim_h = jnp.sum(emb * h_tok, axis=1) / (en * hn + 1e-8)
        sim_t = jnp.sum(emb * t_tok, axis=1) / (en * tn + 1e-8)
        dist = (1.0 - 0.5 * (sim_h + sim_t)) * 0.5
        x = jnp.dot(emb, wcat_ref[...], preferred_element_type=f32)
        c = jnp.maximum(
            dist[:, None] * x[:, :D] + att[:, None] * x[:, D:] + bias, 0.0
        ) * 0.001
        return acc + lax.dot_general(
            onehot, c, (((0,), (0,)), ((), ())), preferred_element_type=f32)

    acc = lax.fori_loop(0, NBLK, pass_b, jnp.zeros((B, D), f32))
    counts = (hi_col_ref[...] - lo_col_ref[...]).astype(f32)
    out_ref[...] = acc / jnp.maximum(counts, 1.0) + tail - head


def kernel(table, w_d_w, w_d_b, w_e_w, w_e_b, eat_w, eat_b,
           flat_ids, cu_seqlens, head_ids, tail_ids):
    T = flat_ids.shape[0]
    B = head_ids.shape[0]

    n_ids = T + 2 * B
    total_rows = -(-n_ids // (NW * GCHUNK)) * (NW * GCHUNK)
    ids_all = jnp.concatenate([
        flat_ids.astype(jnp.int32),
        head_ids.astype(jnp.int32),
        tail_ids.astype(jnp.int32),
        jnp.zeros((total_rows - n_ids,), jnp.int32),
    ])

    gathered = _make_gather(total_rows)(table, ids_all)
    emb = gathered[:T]
    head_e = gathered[T:T + B]
    tail_e = gathered[T + B:T + 2 * B]

    cu = cu_seqlens.astype(jnp.int32)
    lo = cu[:B].reshape(1, B)
    hi = cu[1:B + 1].reshape(1, B)
    lo_col = cu[:B].reshape(B, 1)
    hi_col = cu[1:B + 1].reshape(B, 1)
    wcat = jnp.concatenate([w_d_w, w_e_w], axis=1)
    bias = (w_d_b + w_e_b).reshape(1, D)
    eat_row = eat_w.reshape(1, D)

    out = pl.pallas_call(
        _tc_body,
        out_shape=jax.ShapeDtypeStruct((B, D), jnp.float32),
        scratch_shapes=[pltpu.VMEM((T // 512, 512), jnp.float32)],
    )(emb, head_e, tail_e, lo, hi, lo_col, hi_col, eat_row, wcat, bias)
    return out


# speedup vs baseline: 4.2584x; 1.5319x over previous
import functools

import jax
import jax.numpy as jnp
from jax import lax
from jax.experimental import pallas as pl
from jax.experimental.pallas import tpu as pltpu
from jax.experimental.pallas import tpu_sc as plsc

D = 128
NC = 2
NS = 16
NW = NC * NS
GCHUNK = 96
NBUF = 4


def _make_gather(total_rows: int):
    assert total_rows % (NW * GCHUNK) == 0
    per_w = total_rows // NW
    n = per_w // GCHUNK
    mesh = plsc.VectorSubcoreMesh(core_axis_name="c", subcore_axis_name="s")

    @functools.partial(
        pl.kernel,
        mesh=mesh,
        out_type=jax.ShapeDtypeStruct((total_rows, D), jnp.float32),
        scratch_types=[
            pltpu.VMEM((per_w,), jnp.int32),
            [pltpu.VMEM((GCHUNK, D), jnp.float32) for _ in range(NBUF)],
            [pltpu.SemaphoreType.DMA for _ in range(NBUF)],
            [pltpu.SemaphoreType.DMA for _ in range(NBUF)],
        ],
    )
    def gather_kernel(table_hbm, ids_hbm, out_hbm, idx_v, rows, gsem, wsem):
        wid = lax.axis_index("s") * NC + lax.axis_index("c")
        base = wid * per_w
        pltpu.sync_copy(ids_hbm.at[pl.ds(pl.multiple_of(base, 8), per_w)],
                        idx_v)

        def start_gather(k):
            b = k % NBUF
            return pltpu.async_copy(
                table_hbm.at[idx_v.at[pl.ds(k * GCHUNK, GCHUNK)]],
                rows[b], gsem[b])

        def start_wb(k):
            b = k % NBUF
            off = pl.multiple_of(base + k * GCHUNK, 8)
            return pltpu.async_copy(
                rows[b], out_hbm.at[pl.ds(off, GCHUNK), :], wsem[b])

        gd, wbd = {}, {}
        wb_waited = set()

        def wait_wb(k):
            if k in wbd and k not in wb_waited:
                wbd[k].wait()
                wb_waited.add(k)

        for k in range(min(NBUF - 1, n)):
            gd[k] = start_gather(k)
        for j in range(n):
            k = j + NBUF - 1
            if k < n:
                wait_wb(j - 1)
                gd[k] = start_gather(k)
            gd[j].wait()
            wbd[j] = start_wb(j)
        for j in range(n):
            wait_wb(j)

    return gather_kernel


def _tc_body(g_ref, lo_ref, hi_ref, lo_col_ref, hi_col_ref, eat_ref,
             wcat_ref, bias_ref, out_ref, s_scr):
    f32 = jnp.float32
    B = lo_ref.shape[1]
    BLK = 2048
    T = (g_ref.shape[0] - 2 * B) // BLK * BLK
    NBLK = T // BLK

    lo = lo_ref[...]
    hi = hi_ref[...]
    head = g_ref[T:T + B, :]
    tail = g_ref[T + B:T + 2 * B, :]
    eat = eat_ref[...]
    bias = bias_ref[...]
    idx0 = lax.broadcasted_iota(jnp.int32, (BLK, B), 0)

    def pass_a(i, den):
        emb = g_ref[pl.ds(i * BLK, BLK), :]
        s = jnp.exp(jnp.sum(emb * eat, axis=1))
        s_scr[pl.ds(i, 1), :] = s.reshape(1, BLK)
        pos = idx0 + i * BLK
        onehot = jnp.logical_and(pos >= lo, pos < hi).astype(f32)
        return den + jnp.sum(onehot * s[:, None], axis=0, keepdims=True)

    den = lax.fori_loop(0, NBLK, pass_a, jnp.zeros((1, B), f32))

    def pass_b(i, acc):
        emb = g_ref[pl.ds(i * BLK, BLK), :]
        s = s_scr[pl.ds(i, 1), :].reshape(BLK)
        pos = idx0 + i * BLK
        onehot = jnp.logical_and(pos >= lo, pos < hi).astype(f32)
        att = s / jnp.sum(onehot * den, axis=1)
        h_tok = jnp.dot(onehot, head, preferred_element_type=f32)
        t_tok = jnp.dot(onehot, tail, preferred_element_type=f32)
        en = jnp.sqrt(jnp.sum(emb * emb, axis=1))
        hn = jnp.sqrt(jnp.sum(h_tok * h_tok, axis=1))
        tn = jnp.sqrt(jnp.sum(t_tok * t_tok, axis=1))
        sim_h = jnp.sum(emb * h_tok, axis=1) / (en * hn + 1e-8)
        sim_t = jnp.sum(emb * t_tok, axis=1) / (en * tn + 1e-8)
        dist = (1.0 - 0.5 * (sim_h + sim_t)) * 0.5
        x = jnp.dot(emb, wcat_ref[...], preferred_element_type=f32)
        c = jnp.maximum(
            dist[:, None] * x[:, :D] + att[:, None] * x[:, D:] + bias, 0.0
        ) * 0.001
        return acc + lax.dot_general(
            onehot, c, (((0,), (0,)), ((), ())), preferred_element_type=f32)

    acc = lax.fori_loop(0, NBLK, pass_b, jnp.zeros((B, D), f32))
    counts = (hi_col_ref[...] - lo_col_ref[...]).astype(f32)
    out_ref[...] = acc / jnp.maximum(counts, 1.0) + tail - head


def kernel(table, w_d_w, w_d_b, w_e_w, w_e_b, eat_w, eat_b,
           flat_ids, cu_seqlens, head_ids, tail_ids):
    T = flat_ids.shape[0]
    B = head_ids.shape[0]

    n_ids = T + 2 * B
    total_rows = -(-n_ids // (NW * GCHUNK)) * (NW * GCHUNK)
    ids_all = jnp.concatenate([
        flat_ids.astype(jnp.int32),
        head_ids.astype(jnp.int32),
        tail_ids.astype(jnp.int32),
        jnp.zeros((total_rows - n_ids,), jnp.int32),
    ])

    gathered = _make_gather(total_rows)(table, ids_all)

    cu = cu_seqlens.astype(jnp.int32)
    lo = cu[:B].reshape(1, B)
    hi = cu[1:B + 1].reshape(1, B)
    lo_col = cu[:B].reshape(B, 1)
    hi_col = cu[1:B + 1].reshape(B, 1)
    wcat = jnp.concatenate([w_d_w, w_e_w], axis=1)
    bias = (w_d_b + w_e_b).reshape(1, D)
    eat_row = eat_w.reshape(1, D)

    out = pl.pallas_call(
        _tc_body,
        out_shape=jax.ShapeDtypeStruct((B, D), jnp.float32),
        scratch_shapes=[pltpu.VMEM((T // 2048, 2048), jnp.float32)],
    )(gathered, lo, hi, lo_col, hi_col, eat_row, wcat, bias)
    return out
